# baseline (speedup 1.0000x reference)
import functools

import jax
import jax.numpy as jnp
from jax.experimental import pallas as pl


_BM = 400


def _proj_kernel(x_ref, w_ref, o_ref):
    o_ref[...] = jnp.dot(x_ref[...], w_ref[...],
                         preferred_element_type=jnp.float32)


def _layer_kernel(adj_ref, s_ref, b_ref, w_ref, o_ref):
    h = jnp.dot(adj_ref[...], s_ref[...],
                preferred_element_type=jnp.float32) + b_ref[...]
    h = jnp.maximum(h, 0.0)
    o_ref[...] = jnp.dot(h, w_ref[...], preferred_element_type=jnp.float32)


def _final_kernel(adj_ref, s_ref, b_ref, o_ref):
    h = jnp.dot(adj_ref[...], s_ref[...],
                preferred_element_type=jnp.float32) + b_ref[...]
    o_ref[...] = jax.nn.sigmoid(h)


def _proj(x, w, interpret=False):
    n, f = x.shape
    k = w.shape[1]
    return pl.pallas_call(
        _proj_kernel,
        grid=(n // _BM,),
        in_specs=[
            pl.BlockSpec((_BM, f), lambda i: (i, 0)),
            pl.BlockSpec((f, k), lambda i: (0, 0)),
        ],
        out_specs=pl.BlockSpec((_BM, k), lambda i: (i, 0)),
        out_shape=jax.ShapeDtypeStruct((n, k), jnp.float32),
        interpret=interpret,
    )(x, w)


def _layer(adj, s, b, w_next, interpret=False):
    n, k = s.shape
    k2 = w_next.shape[1]
    return pl.pallas_call(
        _layer_kernel,
        grid=(n // _BM,),
        in_specs=[
            pl.BlockSpec((_BM, n), lambda i: (i, 0)),
            pl.BlockSpec((n, k), lambda i: (0, 0)),
            pl.BlockSpec((1, k), lambda i: (0, 0)),
            pl.BlockSpec((k, k2), lambda i: (0, 0)),
        ],
        out_specs=pl.BlockSpec((_BM, k2), lambda i: (i, 0)),
        out_shape=jax.ShapeDtypeStruct((n, k2), jnp.float32),
        interpret=interpret,
    )(adj, s, b.reshape(1, k), w_next)


def _final(adj, s, b, interpret=False):
    n, k = s.shape
    return pl.pallas_call(
        _final_kernel,
        grid=(n // _BM,),
        in_specs=[
            pl.BlockSpec((_BM, n), lambda i: (i, 0)),
            pl.BlockSpec((n, k), lambda i: (0, 0)),
            pl.BlockSpec((1, k), lambda i: (0, 0)),
        ],
        out_specs=pl.BlockSpec((_BM, k), lambda i: (i, 0)),
        out_shape=jax.ShapeDtypeStruct((n, k), jnp.float32),
        interpret=interpret,
    )(adj, s, b.reshape(1, k))


def kernel(x, adj, W1, b1, W2, b2, W3, b3, interpret=False):
    s1 = _proj(x, W1, interpret)
    s2 = _layer(adj, s1, b1, W2, interpret)
    s3 = _layer(adj, s2, b2, W3, interpret)
    return _final(adj, s3, b3, interpret)
